# Initial kernel scaffold; baseline (speedup 1.0000x reference)
import jax
import jax.numpy as jnp
from jax.experimental import pallas as pl


def kernel(x, Ws, rs):
    raise NotImplementedError("write your pallas kernel here")


# speedup vs baseline: 5.3685x; 5.3685x over previous
import functools

import jax
import jax.numpy as jnp
from jax import lax
from jax.experimental import pallas as pl
from jax.experimental.pallas import tpu as pltpu
from jax.experimental.pallas import tpu_sc as plsc

N_DISTS = 100000
N_BITS = 16
HIDDEN = 2 * N_BITS
D = HIDDEN * N_BITS
B = 16384
NC, NS = 2, 16
NW = NC * NS
BPW = B // NW
CHUNK = 64
NCHUNK = BPW // CHUNK


def _make_gather():
    mesh = plsc.VectorSubcoreMesh(core_axis_name="c", subcore_axis_name="s")

    @functools.partial(
        pl.kernel,
        mesh=mesh,
        out_type=[
            jax.ShapeDtypeStruct((B, D), jnp.float32),
            jax.ShapeDtypeStruct((B, HIDDEN), jnp.float32),
        ],
        scratch_types=[
            pltpu.VMEM((NCHUNK, CHUNK), jnp.int32),
            pltpu.VMEM((2, CHUNK, D), jnp.float32),
            pltpu.VMEM((2, CHUNK, HIDDEN), jnp.float32),
            pltpu.SemaphoreType.DMA,
            pltpu.SemaphoreType.DMA,
        ],
        compiler_params=pltpu.CompilerParams(use_tc_tiling_on_sc=False),
    )
    def gather_kernel(x_hbm, ws_hbm, rs_hbm, w_out, r_out,
                      idx_v, wbuf, rbuf, gsem, osem):
        wid = lax.axis_index("s") * NC + lax.axis_index("c")
        base = wid * BPW
        pltpu.sync_copy(x_hbm.at[wid], idx_v)

        def fire_gather(j):
            slot = j % 2
            return (
                pltpu.async_copy(ws_hbm.at[idx_v.at[j]], wbuf.at[slot], gsem),
                pltpu.async_copy(rs_hbm.at[idx_v.at[j]], rbuf.at[slot], gsem),
            )

        def fire_out(j):
            slot = j % 2
            row = base + j * CHUNK
            return (
                pltpu.async_copy(wbuf.at[slot], w_out.at[pl.ds(row, CHUNK)], osem),
                pltpu.async_copy(rbuf.at[slot], r_out.at[pl.ds(row, CHUNK)], osem),
            )

        g = fire_gather(0)
        prev_out = None
        for j in range(NCHUNK):
            g[0].wait()
            g[1].wait()
            if prev_out is not None:
                prev_out[0].wait()
                prev_out[1].wait()
            if j + 1 < NCHUNK:
                g = fire_gather(j + 1)
            prev_out = fire_out(j)
        prev_out[0].wait()
        prev_out[1].wait()

    return gather_kernel


_gather = _make_gather()


def kernel(x, Ws, rs):
    x32 = x.astype(jnp.int32).reshape(NW, NCHUNK, CHUNK)
    W_flat, r = _gather(x32, Ws.reshape(N_DISTS, D), rs)
    return (W_flat.reshape(B, HIDDEN, N_BITS), r)


# speedup vs baseline: 8.7340x; 1.6269x over previous
import functools

import jax
import jax.numpy as jnp
from jax import lax
from jax.experimental import pallas as pl
from jax.experimental.pallas import tpu as pltpu
from jax.experimental.pallas import tpu_sc as plsc

N_DISTS = 100000
N_BITS = 16
HIDDEN = 2 * N_BITS
D = HIDDEN * N_BITS
B = 16384
NC, NS = 2, 16
NW = NC * NS
FPW = D // NW
SEG = 4096
NSEG = B // SEG
L = 16


def _make_gather():
    mesh = plsc.VectorSubcoreMesh(core_axis_name="c", subcore_axis_name="s")

    @functools.partial(
        pl.kernel,
        mesh=mesh,
        out_type=[
            jax.ShapeDtypeStruct((D, B), jnp.float32),
            jax.ShapeDtypeStruct((HIDDEN, B), jnp.float32),
        ],
        scratch_types=[
            pltpu.VMEM((B,), jnp.int32),
            pltpu.VMEM((N_DISTS,), jnp.float32),
            pltpu.VMEM((SEG,), jnp.float32),
        ],
        compiler_params=pltpu.CompilerParams(needs_layout_passes=False),
    )
    def gather_kernel(x_hbm, wst_hbm, rst_hbm, w_out, r_out,
                      x_v, rowbuf, obuf):
        wid = lax.axis_index("s") * NC + lax.axis_index("c")
        pltpu.sync_copy(x_hbm, x_v)

        def do_row(row, src_t, out_t):
            pltpu.sync_copy(src_t.at[row], rowbuf)
            for seg in range(NSEG):

                def gbody(i, _):
                    off = i * L
                    idx = x_v[pl.ds(seg * SEG + off, L)]
                    obuf[pl.ds(off, L)] = plsc.load_gather(rowbuf, [idx])
                    return 0

                lax.fori_loop(0, SEG // L, gbody, 0)
                pltpu.sync_copy(obuf, out_t.at[row, pl.ds(seg * SEG, SEG)])

        for k in range(FPW):
            do_row(wid * FPW + k, wst_hbm, w_out)
        do_row(wid, rst_hbm, r_out)

    return gather_kernel


_gather = _make_gather()


def kernel(x, Ws, rs):
    Wt = Ws.transpose(1, 2, 0).reshape(D, N_DISTS)
    rt = rs.transpose(1, 0)
    OW, OR = _gather(x.astype(jnp.int32), Wt, rt)
    W = OW.reshape(HIDDEN, N_BITS, B).transpose(2, 0, 1)
    r = OR.transpose(1, 0)
    return (W, r)
